# baseline (speedup 1.0000x reference)
import functools

import jax
import jax.numpy as jnp
from jax import lax
from jax.experimental import pallas as pl
from jax.experimental.pallas import tpu as pltpu
from jax.experimental.pallas import tpu_sc as plsc

N_NODES = 10000
N_EDGES = 320000
NC = 2
NS = 16
NW = NC * NS
B = 128
NCH = 80
E_PAD = NW * NCH * B
ROWS_PER_TILE = 640
N_ACC = NS * ROWS_PER_TILE
ZROWS = 64


def _make_agg(D):
    mesh = plsc.VectorSubcoreMesh(core_axis_name="c", subcore_axis_name="s")

    @functools.partial(
        pl.kernel,
        mesh=mesh,
        out_type=jax.ShapeDtypeStruct((NC, N_ACC, D), jnp.float32),
        compiler_params=pltpu.CompilerParams(use_tc_tiling_on_sc=False),
        scratch_types=[
            pltpu.VMEM((NCH, B), jnp.int32),
            pltpu.VMEM((NCH, B), jnp.int32),
            pltpu.VMEM((B, D), jnp.float32),
            pltpu.VMEM_SHARED((N_ACC, D), jnp.float32),
            pltpu.SemaphoreType.DMA,
        ],
    )
    def agg(src_hbm, dst_hbm, h_hbm, out_hbm, src_v, dst_v, r0, acc, s0):
        c = lax.axis_index("c")
        s = lax.axis_index("s")
        wid = c * NS + s

        def _zb(k, _):
            i = k // (D // 16)
            j = k % (D // 16)
            r0[i, pl.ds(j * 16, 16)] = jnp.zeros((16,), jnp.float32)
            return 0
        lax.fori_loop(0, B * (D // 16), _zb, 0)

        def _zc(i, _):
            pltpu.sync_copy(
                r0.at[pl.ds(0, ZROWS)],
                acc.at[pl.ds(s * ROWS_PER_TILE + i * ZROWS, ZROWS)])
            return 0
        lax.fori_loop(0, ROWS_PER_TILE // ZROWS, _zc, 0)

        pltpu.sync_copy(src_hbm.at[wid], src_v)
        pltpu.sync_copy(dst_hbm.at[wid], dst_v)
        plsc.subcore_barrier()

        def _body(j, _):
            pltpu.sync_copy(h_hbm.at[src_v.at[j]], r0)
            pltpu.sync_copy(r0, acc.at[dst_v.at[j]], add=True)
            return 0
        lax.fori_loop(0, NCH, _body, 0)
        plsc.subcore_barrier()

        pltpu.sync_copy(
            acc.at[pl.ds(s * ROWS_PER_TILE, ROWS_PER_TILE)],
            out_hbm.at[c, pl.ds(s * ROWS_PER_TILE, ROWS_PER_TILE)],
        )

    return agg


def _make_deg():
    D = 16
    mesh = plsc.VectorSubcoreMesh(core_axis_name="c", subcore_axis_name="s")

    @functools.partial(
        pl.kernel,
        mesh=mesh,
        out_type=jax.ShapeDtypeStruct((NC, N_ACC, D), jnp.float32),
        compiler_params=pltpu.CompilerParams(use_tc_tiling_on_sc=False),
        scratch_types=[
            pltpu.VMEM((NCH, B), jnp.int32),
            pltpu.VMEM((B, D), jnp.float32),
            pltpu.VMEM((ZROWS, D), jnp.float32),
            pltpu.VMEM_SHARED((N_ACC, D), jnp.float32),
        ],
    )
    def deg(dst_hbm, out_hbm, dst_v, ones_v, zbuf, acc):
        c = lax.axis_index("c")
        s = lax.axis_index("s")
        wid = c * NS + s

        def _init(i, _):
            ones_v[i] = jnp.ones((D,), jnp.float32)
            return 0
        lax.fori_loop(0, B, _init, 0)

        def _zb(i, _):
            zbuf[i] = jnp.zeros((D,), jnp.float32)
            return 0
        lax.fori_loop(0, ZROWS, _zb, 0)

        def _zc(i, _):
            pltpu.sync_copy(zbuf, acc.at[pl.ds(s * ROWS_PER_TILE + i * ZROWS, ZROWS)])
            return 0
        lax.fori_loop(0, ROWS_PER_TILE // ZROWS, _zc, 0)

        pltpu.sync_copy(dst_hbm.at[wid], dst_v)
        plsc.subcore_barrier()

        def _body(j, _):
            pltpu.sync_copy(ones_v, acc.at[dst_v.at[j]], add=True)
            return 0
        lax.fori_loop(0, NCH, _body, 0)
        plsc.subcore_barrier()

        pltpu.sync_copy(
            acc.at[pl.ds(s * ROWS_PER_TILE, ROWS_PER_TILE)],
            out_hbm.at[c, pl.ds(s * ROWS_PER_TILE, ROWS_PER_TILE)],
        )

    return deg


_BLK = 1000


def _norm_from(d_blk):
    deg = d_blk[:, 0:1] + d_blk[:, 1:2]
    return jnp.where(deg > 0, lax.rsqrt(deg), 0.0)


def _tc_first(feats, W, dpt):
    Dn = W.shape[1]

    def body(f_ref, w_ref, d_ref, o_ref):
        norm = _norm_from(d_ref[...])
        x = f_ref[...] * norm
        o_ref[...] = jnp.dot(x, w_ref[...], preferred_element_type=jnp.float32)

    return pl.pallas_call(
        body,
        grid=(N_NODES // _BLK,),
        in_specs=[
            pl.BlockSpec((_BLK, feats.shape[1]), lambda i: (i, 0)),
            pl.BlockSpec((feats.shape[1], Dn), lambda i: (0, 0)),
            pl.BlockSpec((_BLK, 2), lambda i: (i, 0)),
        ],
        out_specs=pl.BlockSpec((_BLK, Dn), lambda i: (i, 0)),
        out_shape=jax.ShapeDtypeStruct((N_NODES, Dn), jnp.float32),
    )(feats, W, dpt)


def _tc_mid(parts, dpt, b, W):
    D = parts.shape[2]
    Dn = W.shape[1]

    def body(p_ref, d_ref, b_ref, w_ref, o_ref):
        norm = _norm_from(d_ref[...])
        p = p_ref[...]
        x = (p[0] + p[1]) * norm + b_ref[...]
        x = jnp.maximum(x, 0.0) * norm
        o_ref[...] = jnp.dot(x, w_ref[...], preferred_element_type=jnp.float32)

    return pl.pallas_call(
        body,
        grid=(N_NODES // _BLK,),
        in_specs=[
            pl.BlockSpec((NC, _BLK, D), lambda i: (0, i, 0)),
            pl.BlockSpec((_BLK, 2), lambda i: (i, 0)),
            pl.BlockSpec((1, D), lambda i: (0, 0)),
            pl.BlockSpec((D, Dn), lambda i: (0, 0)),
        ],
        out_specs=pl.BlockSpec((_BLK, Dn), lambda i: (i, 0)),
        out_shape=jax.ShapeDtypeStruct((N_NODES, Dn), jnp.float32),
    )(parts, dpt, b, W)


def _tc_last(parts, dpt, b):
    D = parts.shape[2]

    def body(p_ref, d_ref, b_ref, o_ref):
        norm = _norm_from(d_ref[...])
        p = p_ref[...]
        x = (p[0] + p[1]) * norm + b_ref[...]
        o_ref[...] = jnp.tanh(x)

    return pl.pallas_call(
        body,
        grid=(N_NODES // _BLK,),
        in_specs=[
            pl.BlockSpec((NC, _BLK, D), lambda i: (0, i, 0)),
            pl.BlockSpec((_BLK, 2), lambda i: (i, 0)),
            pl.BlockSpec((1, D), lambda i: (0, 0)),
        ],
        out_specs=pl.BlockSpec((_BLK, D), lambda i: (i, 0)),
        out_shape=jax.ShapeDtypeStruct((N_NODES, D), jnp.float32),
    )(parts, dpt, b)


def kernel(features, edge_index, W1, b1, W2, b2, W3, b3):
    src = edge_index[0].astype(jnp.int32)
    dst = edge_index[1].astype(jnp.int32)
    pad = E_PAD - N_EDGES
    src_p = jnp.concatenate([src, jnp.zeros((pad,), jnp.int32)]).reshape(NW, NCH, B)
    dst_p = jnp.concatenate([dst, jnp.full((pad,), N_NODES, jnp.int32)]).reshape(NW, NCH, B)

    deg_parts = _make_deg()(dst_p)
    dpt = deg_parts[:, :N_NODES, 0].T

    agg128 = _make_agg(128)
    agg48 = _make_agg(48)

    hw1 = _tc_first(features, W1, dpt)
    p1 = agg128(src_p, dst_p, hw1)
    hw2 = _tc_mid(p1, dpt, b1.reshape(1, 128), W2)
    p2 = agg128(src_p, dst_p, hw2)
    W3p = jnp.pad(W3, ((0, 0), (0, 8)))
    b3p = jnp.pad(b3, (0, 8)).reshape(1, 48)
    hw3 = _tc_mid(p2, dpt, b2.reshape(1, 128), W3p)
    p3 = agg48(src_p, dst_p, hw3)
    out48 = _tc_last(p3, dpt, b3p)
    return out48[:, :40]


# speedup vs baseline: 1.0016x; 1.0016x over previous
import functools

import jax
import jax.numpy as jnp
from jax import lax
from jax.experimental import pallas as pl
from jax.experimental.pallas import tpu as pltpu
from jax.experimental.pallas import tpu_sc as plsc

N_NODES = 10000
N_EDGES = 320000
NC = 2
NS = 16
NW = NC * NS
B = 128
NCH = 80
E_PAD = NW * NCH * B
ROWS_PER_TILE = 640
N_ACC = NS * ROWS_PER_TILE
ZROWS = 64


def _make_agg(D):
    mesh = plsc.VectorSubcoreMesh(core_axis_name="c", subcore_axis_name="s")

    @functools.partial(
        pl.kernel,
        mesh=mesh,
        out_type=jax.ShapeDtypeStruct((NC, N_ACC, D), jnp.float32),
        compiler_params=pltpu.CompilerParams(use_tc_tiling_on_sc=False),
        scratch_types=[
            pltpu.VMEM((NCH, B), jnp.int32),
            pltpu.VMEM((NCH, B), jnp.int32),
            pltpu.VMEM((B, D), jnp.float32),
            pltpu.VMEM_SHARED((N_ACC, D), jnp.float32),
            pltpu.SemaphoreType.DMA,
        ],
    )
    def agg(src_hbm, dst_hbm, h_hbm, out_hbm, src_v, dst_v, r0, acc, s0):
        c = lax.axis_index("c")
        s = lax.axis_index("s")
        wid = c * NS + s

        def _zb(k, _):
            i = k // (D // 16)
            j = k % (D // 16)
            r0[i, pl.ds(j * 16, 16)] = jnp.zeros((16,), jnp.float32)
            return 0
        lax.fori_loop(0, B * (D // 16), _zb, 0)

        def _zc(i, _):
            pltpu.sync_copy(
                r0.at[pl.ds(0, ZROWS)],
                acc.at[pl.ds(s * ROWS_PER_TILE + i * ZROWS, ZROWS)])
            return 0
        lax.fori_loop(0, ROWS_PER_TILE // ZROWS, _zc, 0)

        pltpu.sync_copy(src_hbm.at[wid], src_v)
        pltpu.sync_copy(dst_hbm.at[wid], dst_v)
        plsc.subcore_barrier()

        def _body(j, _):
            pltpu.sync_copy(h_hbm.at[src_v.at[j]], r0)
            pltpu.sync_copy(r0, acc.at[dst_v.at[j]], add=True)
            return 0
        lax.fori_loop(0, NCH, _body, 0)
        plsc.subcore_barrier()

        pltpu.sync_copy(
            acc.at[pl.ds(s * ROWS_PER_TILE, ROWS_PER_TILE)],
            out_hbm.at[c, pl.ds(s * ROWS_PER_TILE, ROWS_PER_TILE)],
        )

    return agg


def _make_deg():
    D = 16
    mesh = plsc.VectorSubcoreMesh(core_axis_name="c", subcore_axis_name="s")

    @functools.partial(
        pl.kernel,
        mesh=mesh,
        out_type=jax.ShapeDtypeStruct((NC, N_ACC, D), jnp.float32),
        compiler_params=pltpu.CompilerParams(use_tc_tiling_on_sc=False),
        scratch_types=[
            pltpu.VMEM((NCH, B), jnp.int32),
            pltpu.VMEM((B, D), jnp.float32),
            pltpu.VMEM((ZROWS, D), jnp.float32),
            pltpu.VMEM_SHARED((N_ACC, D), jnp.float32),
        ],
    )
    def deg(dst_hbm, out_hbm, dst_v, ones_v, zbuf, acc):
        c = lax.axis_index("c")
        s = lax.axis_index("s")
        wid = c * NS + s

        def _init(i, _):
            ones_v[i] = jnp.ones((D,), jnp.float32)
            return 0
        lax.fori_loop(0, B, _init, 0)

        def _zb(i, _):
            zbuf[i] = jnp.zeros((D,), jnp.float32)
            return 0
        lax.fori_loop(0, ZROWS, _zb, 0)

        def _zc(i, _):
            pltpu.sync_copy(zbuf, acc.at[pl.ds(s * ROWS_PER_TILE + i * ZROWS, ZROWS)])
            return 0
        lax.fori_loop(0, ROWS_PER_TILE // ZROWS, _zc, 0)

        pltpu.sync_copy(dst_hbm.at[wid], dst_v)
        plsc.subcore_barrier()

        def _body(j, _):
            pltpu.sync_copy(ones_v, acc.at[dst_v.at[j]], add=True)
            return 0
        lax.fori_loop(0, NCH, _body, 0)
        plsc.subcore_barrier()

        pltpu.sync_copy(
            acc.at[pl.ds(s * ROWS_PER_TILE, ROWS_PER_TILE)],
            out_hbm.at[c, pl.ds(s * ROWS_PER_TILE, ROWS_PER_TILE)],
        )

    return deg


_BLK = 1000


def _norm_from(d_blk):
    deg = d_blk[:, 0:1] + d_blk[:, 1:2]
    return jnp.where(deg > 0, lax.rsqrt(deg), 0.0)


def _tc_first(feats, W, dpt):
    Dn = W.shape[1]

    def body(f_ref, w_ref, d_ref, o_ref):
        norm = _norm_from(d_ref[...])
        x = f_ref[...] * norm
        o_ref[...] = jnp.dot(x, w_ref[...], preferred_element_type=jnp.float32)

    return pl.pallas_call(
        body,
        grid=(N_NODES // _BLK,),
        in_specs=[
            pl.BlockSpec((_BLK, feats.shape[1]), lambda i: (i, 0)),
            pl.BlockSpec((feats.shape[1], Dn), lambda i: (0, 0)),
            pl.BlockSpec((_BLK, 2), lambda i: (i, 0)),
        ],
        out_specs=pl.BlockSpec((_BLK, Dn), lambda i: (i, 0)),
        out_shape=jax.ShapeDtypeStruct((N_NODES, Dn), jnp.float32),
    )(feats, W, dpt)


def _tc_mid(parts, dpt, b, W):
    D = parts.shape[2]
    Dn = W.shape[1]

    def body(p_ref, d_ref, b_ref, w_ref, o_ref):
        norm = _norm_from(d_ref[...])
        p = p_ref[...]
        x = (p[0] + p[1]) * norm + b_ref[...]
        x = jnp.maximum(x, 0.0) * norm
        o_ref[...] = jnp.dot(x, w_ref[...], preferred_element_type=jnp.float32)

    return pl.pallas_call(
        body,
        grid=(N_NODES // _BLK,),
        in_specs=[
            pl.BlockSpec((NC, _BLK, D), lambda i: (0, i, 0)),
            pl.BlockSpec((_BLK, 2), lambda i: (i, 0)),
            pl.BlockSpec((1, D), lambda i: (0, 0)),
            pl.BlockSpec((D, Dn), lambda i: (0, 0)),
        ],
        out_specs=pl.BlockSpec((_BLK, Dn), lambda i: (i, 0)),
        out_shape=jax.ShapeDtypeStruct((N_NODES, Dn), jnp.float32),
    )(parts, dpt, b, W)


def _tc_last(parts, dpt, b):
    D = parts.shape[2]

    def body(p_ref, d_ref, b_ref, o_ref):
        norm = _norm_from(d_ref[...])
        p = p_ref[...]
        x = (p[0] + p[1]) * norm + b_ref[...]
        o_ref[...] = jnp.tanh(x)

    return pl.pallas_call(
        body,
        grid=(N_NODES // _BLK,),
        in_specs=[
            pl.BlockSpec((NC, _BLK, D), lambda i: (0, i, 0)),
            pl.BlockSpec((_BLK, 2), lambda i: (i, 0)),
            pl.BlockSpec((1, D), lambda i: (0, 0)),
        ],
        out_specs=pl.BlockSpec((_BLK, D), lambda i: (i, 0)),
        out_shape=jax.ShapeDtypeStruct((N_NODES, D), jnp.float32),
    )(parts, dpt, b)


def kernel(features, edge_index, W1, b1, W2, b2, W3, b3):
    src = edge_index[0].astype(jnp.int32)
    dst = edge_index[1].astype(jnp.int32)
    pad = E_PAD - N_EDGES
    src_p = jnp.concatenate([src, jnp.zeros((pad,), jnp.int32)]).reshape(NW, NCH, B)
    trash = N_NODES + (jnp.arange(pad, dtype=jnp.int32) % (N_ACC - N_NODES))
    dst_p = jnp.concatenate([dst, trash]).reshape(NW, NCH, B)

    deg_parts = _make_deg()(dst_p)
    dpt = deg_parts[:, :N_NODES, 0].T

    agg128 = _make_agg(128)
    agg48 = _make_agg(48)

    hw1 = _tc_first(features, W1, dpt)
    p1 = agg128(src_p, dst_p, hw1)
    hw2 = _tc_mid(p1, dpt, b1.reshape(1, 128), W2)
    p2 = agg128(src_p, dst_p, hw2)
    W3p = jnp.pad(W3, ((0, 0), (0, 8)))
    b3p = jnp.pad(b3, (0, 8)).reshape(1, 48)
    hw3 = _tc_mid(p2, dpt, b2.reshape(1, 128), W3p)
    p3 = agg48(src_p, dst_p, hw3)
    out48 = _tc_last(p3, dpt, b3p)
    return out48[:, :40]


# speedup vs baseline: 1.0040x; 1.0025x over previous
import functools

import jax
import jax.numpy as jnp
from jax import lax
from jax.experimental import pallas as pl
from jax.experimental.pallas import tpu as pltpu
from jax.experimental.pallas import tpu_sc as plsc

N_NODES = 10000
N_EDGES = 320000
NC = 2
NS = 16
NW = NC * NS
B = 128
NCH = 80
E_PAD = NW * NCH * B
ROWS_PER_TILE = 640
N_ACC = NS * ROWS_PER_TILE
ZROWS = 64


def _make_agg(D):
    mesh = plsc.VectorSubcoreMesh(core_axis_name="c", subcore_axis_name="s")

    @functools.partial(
        pl.kernel,
        mesh=mesh,
        out_type=jax.ShapeDtypeStruct((NC, N_ACC, D), jnp.float32),
        compiler_params=pltpu.CompilerParams(use_tc_tiling_on_sc=False),
        scratch_types=[
            pltpu.VMEM((NCH, B), jnp.int32),
            pltpu.VMEM((NCH, B), jnp.int32),
            pltpu.VMEM((B, D), jnp.float32),
            pltpu.VMEM((ZROWS, D), jnp.float32),
            pltpu.VMEM_SHARED((N_ACC, D), jnp.float32),
        ],
    )
    def agg(src_hbm, dst_hbm, h_hbm, out_hbm, src_v, dst_v, r0, zbuf, acc):
        c = lax.axis_index("c")
        s = lax.axis_index("s")
        wid = c * NS + s

        def _zb(k, _):
            i = k // (D // 16)
            j = k % (D // 16)
            zbuf[i, pl.ds(j * 16, 16)] = jnp.zeros((16,), jnp.float32)
            return 0
        lax.fori_loop(0, ZROWS * (D // 16), _zb, 0)

        def _zc(i, _):
            pltpu.sync_copy(
                zbuf,
                acc.at[pl.ds(s * ROWS_PER_TILE + i * ZROWS, ZROWS)])
            return 0
        lax.fori_loop(0, ROWS_PER_TILE // ZROWS, _zc, 0)

        pltpu.sync_copy(src_hbm.at[wid], src_v)
        pltpu.sync_copy(dst_hbm.at[wid], dst_v)
        plsc.subcore_barrier()

        def _body(j, _):
            pltpu.sync_copy(h_hbm.at[src_v.at[j]], r0)
            pltpu.sync_copy(r0, acc.at[dst_v.at[j]], add=True)
            return 0
        lax.fori_loop(0, NCH, _body, 0)
        plsc.subcore_barrier()

        pltpu.sync_copy(
            acc.at[pl.ds(s * ROWS_PER_TILE, ROWS_PER_TILE)],
            out_hbm.at[c, pl.ds(s * ROWS_PER_TILE, ROWS_PER_TILE)],
        )

    return agg


def _make_deg():
    D = 16
    mesh = plsc.VectorSubcoreMesh(core_axis_name="c", subcore_axis_name="s")

    @functools.partial(
        pl.kernel,
        mesh=mesh,
        out_type=jax.ShapeDtypeStruct((NC, N_ACC, D), jnp.float32),
        compiler_params=pltpu.CompilerParams(use_tc_tiling_on_sc=False),
        scratch_types=[
            pltpu.VMEM((NCH, B), jnp.int32),
            pltpu.VMEM((B, D), jnp.float32),
            pltpu.VMEM((ZROWS, D), jnp.float32),
            pltpu.VMEM_SHARED((N_ACC, D), jnp.float32),
        ],
    )
    def deg(dst_hbm, out_hbm, dst_v, ones_v, zbuf, acc):
        c = lax.axis_index("c")
        s = lax.axis_index("s")
        wid = c * NS + s

        def _init(i, _):
            ones_v[i] = jnp.ones((D,), jnp.float32)
            return 0
        lax.fori_loop(0, B, _init, 0)

        def _zb(i, _):
            zbuf[i] = jnp.zeros((D,), jnp.float32)
            return 0
        lax.fori_loop(0, ZROWS, _zb, 0)

        def _zc(i, _):
            pltpu.sync_copy(zbuf, acc.at[pl.ds(s * ROWS_PER_TILE + i * ZROWS, ZROWS)])
            return 0
        lax.fori_loop(0, ROWS_PER_TILE // ZROWS, _zc, 0)

        pltpu.sync_copy(dst_hbm.at[wid], dst_v)
        plsc.subcore_barrier()

        def _body(j, _):
            pltpu.sync_copy(ones_v, acc.at[dst_v.at[j]], add=True)
            return 0
        lax.fori_loop(0, NCH, _body, 0)
        plsc.subcore_barrier()

        pltpu.sync_copy(
            acc.at[pl.ds(s * ROWS_PER_TILE, ROWS_PER_TILE)],
            out_hbm.at[c, pl.ds(s * ROWS_PER_TILE, ROWS_PER_TILE)],
        )

    return deg


_BLK = 1000


def _norm_from(d_blk):
    deg = d_blk[:, 0:1] + d_blk[:, 1:2]
    return jnp.where(deg > 0, lax.rsqrt(deg), 0.0)


def _tc_first(feats, W, dpt):
    Dn = W.shape[1]

    def body(f_ref, w_ref, d_ref, o_ref):
        norm = _norm_from(d_ref[...])
        x = f_ref[...] * norm
        o_ref[...] = jnp.dot(x, w_ref[...], preferred_element_type=jnp.float32)

    return pl.pallas_call(
        body,
        grid=(N_NODES // _BLK,),
        in_specs=[
            pl.BlockSpec((_BLK, feats.shape[1]), lambda i: (i, 0)),
            pl.BlockSpec((feats.shape[1], Dn), lambda i: (0, 0)),
            pl.BlockSpec((_BLK, 2), lambda i: (i, 0)),
        ],
        out_specs=pl.BlockSpec((_BLK, Dn), lambda i: (i, 0)),
        out_shape=jax.ShapeDtypeStruct((N_NODES, Dn), jnp.float32),
    )(feats, W, dpt)


def _tc_mid(parts, dpt, b, W):
    D = parts.shape[2]
    Dn = W.shape[1]

    def body(p_ref, d_ref, b_ref, w_ref, o_ref):
        norm = _norm_from(d_ref[...])
        p = p_ref[...]
        x = (p[0] + p[1]) * norm + b_ref[...]
        x = jnp.maximum(x, 0.0) * norm
        o_ref[...] = jnp.dot(x, w_ref[...], preferred_element_type=jnp.float32)

    return pl.pallas_call(
        body,
        grid=(N_NODES // _BLK,),
        in_specs=[
            pl.BlockSpec((NC, _BLK, D), lambda i: (0, i, 0)),
            pl.BlockSpec((_BLK, 2), lambda i: (i, 0)),
            pl.BlockSpec((1, D), lambda i: (0, 0)),
            pl.BlockSpec((D, Dn), lambda i: (0, 0)),
        ],
        out_specs=pl.BlockSpec((_BLK, Dn), lambda i: (i, 0)),
        out_shape=jax.ShapeDtypeStruct((N_NODES, Dn), jnp.float32),
    )(parts, dpt, b, W)


def _tc_last(parts, dpt, b):
    D = parts.shape[2]

    def body(p_ref, d_ref, b_ref, o_ref):
        norm = _norm_from(d_ref[...])
        p = p_ref[...]
        x = (p[0] + p[1]) * norm + b_ref[...]
        o_ref[...] = jnp.tanh(x)

    return pl.pallas_call(
        body,
        grid=(N_NODES // _BLK,),
        in_specs=[
            pl.BlockSpec((NC, _BLK, D), lambda i: (0, i, 0)),
            pl.BlockSpec((_BLK, 2), lambda i: (i, 0)),
            pl.BlockSpec((1, D), lambda i: (0, 0)),
        ],
        out_specs=pl.BlockSpec((_BLK, D), lambda i: (i, 0)),
        out_shape=jax.ShapeDtypeStruct((N_NODES, D), jnp.float32),
    )(parts, dpt, b)


def kernel(features, edge_index, W1, b1, W2, b2, W3, b3):
    src = edge_index[0].astype(jnp.int32)
    dst = edge_index[1].astype(jnp.int32)
    pad = E_PAD - N_EDGES
    src_p = jnp.concatenate([src, jnp.zeros((pad,), jnp.int32)]).reshape(NW, NCH, B)
    trash = N_NODES + (jnp.arange(pad, dtype=jnp.int32) % (N_ACC - N_NODES))
    dst_p = jnp.concatenate([dst, trash]).reshape(NW, NCH, B)

    deg_parts = _make_deg()(dst_p)
    dpt = deg_parts[:, :N_NODES, 0].T

    agg128 = _make_agg(128)
    agg48 = _make_agg(48)

    hw1 = _tc_first(features, W1, dpt)
    p1 = agg128(src_p, dst_p, hw1)
    hw2 = _tc_mid(p1, dpt, b1.reshape(1, 128), W2)
    p2 = agg128(src_p, dst_p, hw2)
    W3p = jnp.pad(W3, ((0, 0), (0, 8)))
    b3p = jnp.pad(b3, (0, 8)).reshape(1, 48)
    hw3 = _tc_mid(p2, dpt, b2.reshape(1, 128), W3p)
    p3 = agg48(src_p, dst_p, hw3)
    out48 = _tc_last(p3, dpt, b3p)
    return out48[:, :40]


# speedup vs baseline: 1.5781x; 1.5718x over previous
import functools

import jax
import jax.numpy as jnp
from jax import lax
from jax.experimental import pallas as pl
from jax.experimental.pallas import tpu as pltpu
from jax.experimental.pallas import tpu_sc as plsc

N_NODES = 10000
N_EDGES = 320000
NC = 2
NS = 16
NW = NC * NS
B = 128
NCH = 79
E_PAD = NW * NCH * B
ROWS_PER_TILE = 640
N_ACC = NS * ROWS_PER_TILE
ZROWS = 64


def _make_agg(D):
    mesh = plsc.VectorSubcoreMesh(core_axis_name="c", subcore_axis_name="s")

    @functools.partial(
        pl.kernel,
        mesh=mesh,
        out_type=jax.ShapeDtypeStruct((NC, N_ACC, D), jnp.float32),
        compiler_params=pltpu.CompilerParams(use_tc_tiling_on_sc=False),
        scratch_types=[
            pltpu.VMEM((NCH, B), jnp.int32),
            pltpu.VMEM((NCH, B), jnp.int32),
            pltpu.VMEM((B, D), jnp.float32),
            pltpu.VMEM((ZROWS, D), jnp.float32),
            pltpu.VMEM_SHARED((N_ACC, D), jnp.float32),
        ],
    )
    def agg(src_hbm, dst_hbm, h_hbm, out_hbm, src_v, dst_v, r0, zbuf, acc):
        c = lax.axis_index("c")
        s = lax.axis_index("s")
        wid = c * NS + s

        def _zb(k, _):
            i = k // (D // 16)
            j = k % (D // 16)
            zbuf[i, pl.ds(j * 16, 16)] = jnp.zeros((16,), jnp.float32)
            return 0
        lax.fori_loop(0, ZROWS * (D // 16), _zb, 0)

        def _zc(i, _):
            pltpu.sync_copy(
                zbuf,
                acc.at[pl.ds(s * ROWS_PER_TILE + i * ZROWS, ZROWS)])
            return 0
        lax.fori_loop(0, ROWS_PER_TILE // ZROWS, _zc, 0)

        pltpu.sync_copy(src_hbm.at[wid], src_v)
        pltpu.sync_copy(dst_hbm.at[wid], dst_v)
        plsc.subcore_barrier()

        def _body(j, _):
            pltpu.sync_copy(h_hbm.at[src_v.at[j]], r0)
            pltpu.sync_copy(r0, acc.at[dst_v.at[j]], add=True)
            return 0
        lax.fori_loop(0, NCH, _body, 0)
        plsc.subcore_barrier()

        pltpu.sync_copy(
            acc.at[pl.ds(s * ROWS_PER_TILE, ROWS_PER_TILE)],
            out_hbm.at[c, pl.ds(s * ROWS_PER_TILE, ROWS_PER_TILE)],
        )

    return agg


def _make_deg():
    D = 16
    mesh = plsc.VectorSubcoreMesh(core_axis_name="c", subcore_axis_name="s")

    @functools.partial(
        pl.kernel,
        mesh=mesh,
        out_type=jax.ShapeDtypeStruct((NC, N_ACC, D), jnp.float32),
        compiler_params=pltpu.CompilerParams(use_tc_tiling_on_sc=False),
        scratch_types=[
            pltpu.VMEM((NCH, B), jnp.int32),
            pltpu.VMEM((B, D), jnp.float32),
            pltpu.VMEM((ZROWS, D), jnp.float32),
            pltpu.VMEM_SHARED((N_ACC, D), jnp.float32),
        ],
    )
    def deg(dst_hbm, out_hbm, dst_v, ones_v, zbuf, acc):
        c = lax.axis_index("c")
        s = lax.axis_index("s")
        wid = c * NS + s

        def _init(i, _):
            ones_v[i] = jnp.ones((D,), jnp.float32)
            return 0
        lax.fori_loop(0, B, _init, 0)

        def _zb(i, _):
            zbuf[i] = jnp.zeros((D,), jnp.float32)
            return 0
        lax.fori_loop(0, ZROWS, _zb, 0)

        def _zc(i, _):
            pltpu.sync_copy(zbuf, acc.at[pl.ds(s * ROWS_PER_TILE + i * ZROWS, ZROWS)])
            return 0
        lax.fori_loop(0, ROWS_PER_TILE // ZROWS, _zc, 0)

        pltpu.sync_copy(dst_hbm.at[wid], dst_v)
        plsc.subcore_barrier()

        def _body(j, _):
            pltpu.sync_copy(ones_v, acc.at[dst_v.at[j]], add=True)
            return 0
        lax.fori_loop(0, NCH, _body, 0)
        plsc.subcore_barrier()

        pltpu.sync_copy(
            acc.at[pl.ds(s * ROWS_PER_TILE, ROWS_PER_TILE)],
            out_hbm.at[c, pl.ds(s * ROWS_PER_TILE, ROWS_PER_TILE)],
        )

    return deg


_BLK = 1000


def _norm_from(d_blk):
    deg = d_blk[:, 0:1] + d_blk[:, 1:2]
    return jnp.where(deg > 0, lax.rsqrt(deg), 0.0)


def _tc_first(feats, W, dpt):
    Dn = W.shape[1]

    def body(f_ref, w_ref, d_ref, o_ref):
        norm = _norm_from(d_ref[...])
        x = f_ref[...] * norm
        o_ref[...] = jnp.dot(x, w_ref[...], preferred_element_type=jnp.float32)

    return pl.pallas_call(
        body,
        grid=(N_NODES // _BLK,),
        in_specs=[
            pl.BlockSpec((_BLK, feats.shape[1]), lambda i: (i, 0)),
            pl.BlockSpec((feats.shape[1], Dn), lambda i: (0, 0)),
            pl.BlockSpec((_BLK, 2), lambda i: (i, 0)),
        ],
        out_specs=pl.BlockSpec((_BLK, Dn), lambda i: (i, 0)),
        out_shape=jax.ShapeDtypeStruct((N_NODES, Dn), jnp.float32),
    )(feats, W, dpt)


def _tc_mid(parts, dpt, b, W):
    D = parts.shape[2]
    Dn = W.shape[1]

    def body(p_ref, d_ref, b_ref, w_ref, o_ref):
        norm = _norm_from(d_ref[...])
        p = p_ref[...]
        x = (p[0] + p[1]) * norm + b_ref[...]
        x = jnp.maximum(x, 0.0) * norm
        o_ref[...] = jnp.dot(x, w_ref[...], preferred_element_type=jnp.float32)

    return pl.pallas_call(
        body,
        grid=(N_NODES // _BLK,),
        in_specs=[
            pl.BlockSpec((NC, _BLK, D), lambda i: (0, i, 0)),
            pl.BlockSpec((_BLK, 2), lambda i: (i, 0)),
            pl.BlockSpec((1, D), lambda i: (0, 0)),
            pl.BlockSpec((D, Dn), lambda i: (0, 0)),
        ],
        out_specs=pl.BlockSpec((_BLK, Dn), lambda i: (i, 0)),
        out_shape=jax.ShapeDtypeStruct((N_NODES, Dn), jnp.float32),
    )(parts, dpt, b, W)


def _tc_last(parts, dpt, b):
    D = parts.shape[2]

    def body(p_ref, d_ref, b_ref, o_ref):
        norm = _norm_from(d_ref[...])
        p = p_ref[...]
        x = (p[0] + p[1]) * norm + b_ref[...]
        o_ref[...] = jnp.tanh(x)

    return pl.pallas_call(
        body,
        grid=(N_NODES // _BLK,),
        in_specs=[
            pl.BlockSpec((NC, _BLK, D), lambda i: (0, i, 0)),
            pl.BlockSpec((_BLK, 2), lambda i: (i, 0)),
            pl.BlockSpec((1, D), lambda i: (0, 0)),
        ],
        out_specs=pl.BlockSpec((_BLK, D), lambda i: (i, 0)),
        out_shape=jax.ShapeDtypeStruct((N_NODES, D), jnp.float32),
    )(parts, dpt, b)


def kernel(features, edge_index, W1, b1, W2, b2, W3, b3):
    src = edge_index[0].astype(jnp.int32)
    dst = edge_index[1].astype(jnp.int32)
    pad = E_PAD - N_EDGES
    src_p = jnp.concatenate([src, jnp.zeros((pad,), jnp.int32)]).reshape(NW, NCH, B)
    trash = N_NODES + (jnp.arange(pad, dtype=jnp.int32) % (N_ACC - N_NODES))
    dst_p = jnp.concatenate([dst, trash]).reshape(NW, NCH, B)

    deg_parts = _make_deg()(dst_p)
    dpt = deg_parts[:, :N_NODES, 0].T

    agg128 = _make_agg(128)
    agg48 = _make_agg(48)

    hw1 = _tc_first(features, W1, dpt)
    p1 = agg128(src_p, dst_p, hw1)
    hw2 = _tc_mid(p1, dpt, b1.reshape(1, 128), W2)
    p2 = agg128(src_p, dst_p, hw2)
    W3p = jnp.pad(W3, ((0, 0), (0, 8)))
    b3p = jnp.pad(b3, (0, 8)).reshape(1, 48)
    hw3 = _tc_mid(p2, dpt, b2.reshape(1, 128), W3p)
    p3 = agg48(src_p, dst_p, hw3)
    out48 = _tc_last(p3, dpt, b3p)
    return out48[:, :40]


# speedup vs baseline: 2.2816x; 1.4458x over previous
import functools

import jax
import jax.numpy as jnp
from jax import lax
from jax.experimental import pallas as pl
from jax.experimental.pallas import tpu as pltpu
from jax.experimental.pallas import tpu_sc as plsc

N_NODES = 10000
N_EDGES = 320000
NC = 2
NS = 16
NW = NC * NS
B = 128
NCH = 79
E_PAD = NW * NCH * B
ROWS_PER_TILE = 640
N_ACC = NS * ROWS_PER_TILE
ZROWS = 64


def _make_agg(D):
    mesh = plsc.VectorSubcoreMesh(core_axis_name="c", subcore_axis_name="s")

    @functools.partial(
        pl.kernel,
        mesh=mesh,
        out_type=jax.ShapeDtypeStruct((NC, N_ACC, D), jnp.float32),
        compiler_params=pltpu.CompilerParams(use_tc_tiling_on_sc=False),
        scratch_types=[
            pltpu.VMEM((NCH, B), jnp.int32),
            pltpu.VMEM((NCH, B), jnp.int32),
            pltpu.VMEM((B, D), jnp.float32),
            pltpu.VMEM((ZROWS, D), jnp.float32),
            pltpu.VMEM_SHARED((N_ACC, D), jnp.float32),
        ],
    )
    def agg(src_hbm, dst_hbm, h_hbm, out_hbm, src_v, dst_v, r0, zbuf, acc):
        c = lax.axis_index("c")
        s = lax.axis_index("s")
        wid = c * NS + s

        def _zb(k, _):
            i = k // (D // 16)
            j = k % (D // 16)
            zbuf[i, pl.ds(j * 16, 16)] = jnp.zeros((16,), jnp.float32)
            return 0
        lax.fori_loop(0, ZROWS * (D // 16), _zb, 0)

        def _zc(i, _):
            pltpu.sync_copy(
                zbuf,
                acc.at[pl.ds(s * ROWS_PER_TILE + i * ZROWS, ZROWS)])
            return 0
        lax.fori_loop(0, ROWS_PER_TILE // ZROWS, _zc, 0)

        pltpu.sync_copy(src_hbm.at[wid], src_v)
        pltpu.sync_copy(dst_hbm.at[wid], dst_v)
        plsc.subcore_barrier()

        def _body(j, _):
            pltpu.sync_copy(h_hbm.at[src_v.at[j]], r0)
            pltpu.sync_copy(r0, acc.at[dst_v.at[j]], add=True)
            return 0
        lax.fori_loop(0, NCH, _body, 0)
        plsc.subcore_barrier()

        pltpu.sync_copy(
            acc.at[pl.ds(s * ROWS_PER_TILE, ROWS_PER_TILE)],
            out_hbm.at[c, pl.ds(s * ROWS_PER_TILE, ROWS_PER_TILE)],
        )

    return agg


def _make_deg():
    D = 16
    mesh = plsc.VectorSubcoreMesh(core_axis_name="c", subcore_axis_name="s")

    @functools.partial(
        pl.kernel,
        mesh=mesh,
        out_type=jax.ShapeDtypeStruct((NC, N_ACC, D), jnp.float32),
        compiler_params=pltpu.CompilerParams(use_tc_tiling_on_sc=False),
        scratch_types=[
            pltpu.VMEM((NCH, B), jnp.int32),
            pltpu.VMEM((B, D), jnp.float32),
            pltpu.VMEM((ZROWS, D), jnp.float32),
            pltpu.VMEM_SHARED((N_ACC, D), jnp.float32),
        ],
    )
    def deg(dst_hbm, out_hbm, dst_v, ones_v, zbuf, acc):
        c = lax.axis_index("c")
        s = lax.axis_index("s")
        wid = c * NS + s

        def _init(i, _):
            ones_v[i] = jnp.ones((D,), jnp.float32)
            return 0
        lax.fori_loop(0, B, _init, 0)

        def _zb(i, _):
            zbuf[i] = jnp.zeros((D,), jnp.float32)
            return 0
        lax.fori_loop(0, ZROWS, _zb, 0)

        def _zc(i, _):
            pltpu.sync_copy(zbuf, acc.at[pl.ds(s * ROWS_PER_TILE + i * ZROWS, ZROWS)])
            return 0
        lax.fori_loop(0, ROWS_PER_TILE // ZROWS, _zc, 0)

        pltpu.sync_copy(dst_hbm.at[wid], dst_v)
        plsc.subcore_barrier()

        def _body(j, _):
            pltpu.sync_copy(ones_v, acc.at[dst_v.at[j]], add=True)
            return 0
        lax.fori_loop(0, NCH, _body, 0)
        plsc.subcore_barrier()

        pltpu.sync_copy(
            acc.at[pl.ds(s * ROWS_PER_TILE, ROWS_PER_TILE)],
            out_hbm.at[c, pl.ds(s * ROWS_PER_TILE, ROWS_PER_TILE)],
        )

    return deg


_BLK = 1000


def _norm_from(d_blk):
    deg = d_blk[:, 0:1] + d_blk[:, 1:2]
    return jnp.where(deg > 0, lax.rsqrt(deg), 0.0)


def _tc_first(feats, W, dpt):
    Dn = W.shape[1]

    def body(f_ref, w_ref, d_ref, o_ref):
        norm = _norm_from(d_ref[...])
        x = f_ref[...] * norm
        o_ref[...] = jnp.dot(x, w_ref[...], preferred_element_type=jnp.float32)

    return pl.pallas_call(
        body,
        grid=(N_NODES // _BLK,),
        in_specs=[
            pl.BlockSpec((_BLK, feats.shape[1]), lambda i: (i, 0)),
            pl.BlockSpec((feats.shape[1], Dn), lambda i: (0, 0)),
            pl.BlockSpec((_BLK, 2), lambda i: (i, 0)),
        ],
        out_specs=pl.BlockSpec((_BLK, Dn), lambda i: (i, 0)),
        out_shape=jax.ShapeDtypeStruct((N_NODES, Dn), jnp.float32),
    )(feats, W, dpt)


def _tc_mid(parts, dpt, b, W):
    D = parts.shape[2]
    Dn = W.shape[1]

    def body(p_ref, d_ref, b_ref, w_ref, o_ref):
        norm = _norm_from(d_ref[...])
        p = p_ref[...]
        x = (p[0] + p[1]) * norm + b_ref[...]
        x = jnp.maximum(x, 0.0) * norm
        o_ref[...] = jnp.dot(x, w_ref[...], preferred_element_type=jnp.float32)

    return pl.pallas_call(
        body,
        grid=(N_NODES // _BLK,),
        in_specs=[
            pl.BlockSpec((NC, _BLK, D), lambda i: (0, i, 0)),
            pl.BlockSpec((_BLK, 2), lambda i: (i, 0)),
            pl.BlockSpec((1, D), lambda i: (0, 0)),
            pl.BlockSpec((D, Dn), lambda i: (0, 0)),
        ],
        out_specs=pl.BlockSpec((_BLK, Dn), lambda i: (i, 0)),
        out_shape=jax.ShapeDtypeStruct((N_NODES, Dn), jnp.float32),
    )(parts, dpt, b, W)


def _tc_last(parts, dpt, b):
    D = parts.shape[2]

    def body(p_ref, d_ref, b_ref, o_ref):
        norm = _norm_from(d_ref[...])
        p = p_ref[...]
        x = (p[0] + p[1]) * norm + b_ref[...]
        o_ref[...] = jnp.tanh(x)

    return pl.pallas_call(
        body,
        grid=(N_NODES // _BLK,),
        in_specs=[
            pl.BlockSpec((NC, _BLK, D), lambda i: (0, i, 0)),
            pl.BlockSpec((_BLK, 2), lambda i: (i, 0)),
            pl.BlockSpec((1, D), lambda i: (0, 0)),
        ],
        out_specs=pl.BlockSpec((_BLK, D), lambda i: (i, 0)),
        out_shape=jax.ShapeDtypeStruct((N_NODES, D), jnp.float32),
    )(parts, dpt, b)


def kernel(features, edge_index, W1, b1, W2, b2, W3, b3):
    src = edge_index[0].astype(jnp.int32)
    dst = edge_index[1].astype(jnp.int32)
    pad = E_PAD - N_EDGES
    ar = jnp.arange(pad, dtype=jnp.int32)
    src_p = jnp.concatenate([src, ar % N_NODES]).reshape(NW, NCH, B)
    trash = N_NODES + (ar % (N_ACC - N_NODES))
    dst_p = jnp.concatenate([dst, trash]).reshape(NW, NCH, B)

    deg_parts = _make_deg()(dst_p)
    dpt = deg_parts[:, :N_NODES, 0].T

    agg128 = _make_agg(128)
    agg48 = _make_agg(48)

    hw1 = _tc_first(features, W1, dpt)
    p1 = agg128(src_p, dst_p, hw1)
    hw2 = _tc_mid(p1, dpt, b1.reshape(1, 128), W2)
    p2 = agg128(src_p, dst_p, hw2)
    W3p = jnp.pad(W3, ((0, 0), (0, 8)))
    b3p = jnp.pad(b3, (0, 8)).reshape(1, 48)
    hw3 = _tc_mid(p2, dpt, b2.reshape(1, 128), W3p)
    p3 = agg48(src_p, dst_p, hw3)
    out48 = _tc_last(p3, dpt, b3p)
    return out48[:, :40]


# speedup vs baseline: 3.1114x; 1.3637x over previous
import functools

import jax
import jax.numpy as jnp
from jax import lax
from jax.experimental import pallas as pl
from jax.experimental.pallas import tpu as pltpu
from jax.experimental.pallas import tpu_sc as plsc

N_NODES = 10000
N_EDGES = 320000
NC = 2
NS = 16
NW = NC * NS
B = 128
NCH = 80
NCHH = NCH // 2
E_PAD = NW * NCH * B
ROWS_PER_TILE = 640
N_ACC = NS * ROWS_PER_TILE
ZROWS = 64


def _make_agg(D):
    mesh = plsc.VectorSubcoreMesh(core_axis_name="c", subcore_axis_name="s")

    @functools.partial(
        pl.kernel,
        mesh=mesh,
        out_type=jax.ShapeDtypeStruct((NC, N_ACC, D), jnp.float32),
        compiler_params=pltpu.CompilerParams(use_tc_tiling_on_sc=False),
        scratch_types=[
            pltpu.VMEM((NCHH, B), jnp.int32),
            pltpu.VMEM((NCHH, B), jnp.int32),
            pltpu.VMEM((B, D), jnp.float32),
            pltpu.VMEM((B, D), jnp.float32),
            pltpu.VMEM_SHARED((N_ACC, D), jnp.float32),
            pltpu.SemaphoreType.DMA,
            pltpu.SemaphoreType.DMA,
        ],
    )
    def agg(src_hbm, dst_hbm, h_hbm, out_hbm, src_v, dst_v, r0, r1, acc,
            s0, s1):
        c = lax.axis_index("c")
        s = lax.axis_index("s")
        wid = c * NS + s

        def _zb(k, _):
            i = k // (D // 16)
            j = k % (D // 16)
            r0[i, pl.ds(j * 16, 16)] = jnp.zeros((16,), jnp.float32)
            return 0
        lax.fori_loop(0, ZROWS * (D // 16), _zb, 0)

        def _zc(i, _):
            pltpu.sync_copy(
                r0.at[pl.ds(0, ZROWS)],
                acc.at[pl.ds(s * ROWS_PER_TILE + i * ZROWS, ZROWS)])
            return 0
        lax.fori_loop(0, ROWS_PER_TILE // ZROWS, _zc, 0)
        plsc.subcore_barrier()

        def _half(hf, _):
            pltpu.sync_copy(src_hbm.at[wid, pl.ds(hf * NCHH, NCHH)], src_v)
            pltpu.sync_copy(dst_hbm.at[wid, pl.ds(hf * NCHH, NCHH)], dst_v)
            pltpu.async_copy(h_hbm.at[src_v.at[0]], r0, s0)

            def _pair(i, _):
                j0 = 2 * i
                j1 = j0 + 1
                pltpu.async_copy(h_hbm.at[src_v.at[j1]], r1, s1)
                pltpu.make_async_copy(h_hbm.at[src_v.at[j0]], r0, s0).wait()
                pltpu.sync_copy(r0, acc.at[dst_v.at[j0]], add=True)

                @pl.when(j0 + 2 < NCHH)
                def _():
                    pltpu.async_copy(h_hbm.at[src_v.at[j0 + 2]], r0, s0)

                pltpu.make_async_copy(h_hbm.at[src_v.at[j1]], r1, s1).wait()
                pltpu.sync_copy(r1, acc.at[dst_v.at[j1]], add=True)
                return 0
            lax.fori_loop(0, NCHH // 2, _pair, 0)
            return 0
        lax.fori_loop(0, 2, _half, 0)
        plsc.subcore_barrier()

        pltpu.sync_copy(
            acc.at[pl.ds(s * ROWS_PER_TILE, ROWS_PER_TILE)],
            out_hbm.at[c, pl.ds(s * ROWS_PER_TILE, ROWS_PER_TILE)],
        )

    return agg


def _make_deg():
    D = 16
    mesh = plsc.VectorSubcoreMesh(core_axis_name="c", subcore_axis_name="s")

    @functools.partial(
        pl.kernel,
        mesh=mesh,
        out_type=jax.ShapeDtypeStruct((NC, N_ACC, D), jnp.float32),
        compiler_params=pltpu.CompilerParams(use_tc_tiling_on_sc=False),
        scratch_types=[
            pltpu.VMEM((NCH, B), jnp.int32),
            pltpu.VMEM((B, D), jnp.float32),
            pltpu.VMEM((ZROWS, D), jnp.float32),
            pltpu.VMEM_SHARED((N_ACC, D), jnp.float32),
        ],
    )
    def deg(dst_hbm, out_hbm, dst_v, ones_v, zbuf, acc):
        c = lax.axis_index("c")
        s = lax.axis_index("s")
        wid = c * NS + s

        def _init(i, _):
            ones_v[i] = jnp.ones((D,), jnp.float32)
            return 0
        lax.fori_loop(0, B, _init, 0)

        def _zb(i, _):
            zbuf[i] = jnp.zeros((D,), jnp.float32)
            return 0
        lax.fori_loop(0, ZROWS, _zb, 0)

        def _zc(i, _):
            pltpu.sync_copy(zbuf, acc.at[pl.ds(s * ROWS_PER_TILE + i * ZROWS, ZROWS)])
            return 0
        lax.fori_loop(0, ROWS_PER_TILE // ZROWS, _zc, 0)

        pltpu.sync_copy(dst_hbm.at[wid], dst_v)
        plsc.subcore_barrier()

        def _body(j, _):
            pltpu.sync_copy(ones_v, acc.at[dst_v.at[j]], add=True)
            return 0
        lax.fori_loop(0, NCH, _body, 0)
        plsc.subcore_barrier()

        pltpu.sync_copy(
            acc.at[pl.ds(s * ROWS_PER_TILE, ROWS_PER_TILE)],
            out_hbm.at[c, pl.ds(s * ROWS_PER_TILE, ROWS_PER_TILE)],
        )

    return deg


_BLK = 1000


def _norm_from(d_blk):
    deg = d_blk[:, 0:1] + d_blk[:, 1:2]
    return jnp.where(deg > 0, lax.rsqrt(deg), 0.0)


def _tc_first(feats, W, dpt):
    Dn = W.shape[1]

    def body(f_ref, w_ref, d_ref, o_ref):
        norm = _norm_from(d_ref[...])
        x = f_ref[...] * norm
        o_ref[...] = jnp.dot(x, w_ref[...], preferred_element_type=jnp.float32)

    return pl.pallas_call(
        body,
        grid=(N_NODES // _BLK,),
        in_specs=[
            pl.BlockSpec((_BLK, feats.shape[1]), lambda i: (i, 0)),
            pl.BlockSpec((feats.shape[1], Dn), lambda i: (0, 0)),
            pl.BlockSpec((_BLK, 2), lambda i: (i, 0)),
        ],
        out_specs=pl.BlockSpec((_BLK, Dn), lambda i: (i, 0)),
        out_shape=jax.ShapeDtypeStruct((N_NODES, Dn), jnp.float32),
    )(feats, W, dpt)


def _tc_mid(parts, dpt, b, W):
    D = parts.shape[2]
    Dn = W.shape[1]

    def body(p_ref, d_ref, b_ref, w_ref, o_ref):
        norm = _norm_from(d_ref[...])
        p = p_ref[...]
        x = (p[0] + p[1]) * norm + b_ref[...]
        x = jnp.maximum(x, 0.0) * norm
        o_ref[...] = jnp.dot(x, w_ref[...], preferred_element_type=jnp.float32)

    return pl.pallas_call(
        body,
        grid=(N_NODES // _BLK,),
        in_specs=[
            pl.BlockSpec((NC, _BLK, D), lambda i: (0, i, 0)),
            pl.BlockSpec((_BLK, 2), lambda i: (i, 0)),
            pl.BlockSpec((1, D), lambda i: (0, 0)),
            pl.BlockSpec((D, Dn), lambda i: (0, 0)),
        ],
        out_specs=pl.BlockSpec((_BLK, Dn), lambda i: (i, 0)),
        out_shape=jax.ShapeDtypeStruct((N_NODES, Dn), jnp.float32),
    )(parts, dpt, b, W)


def _tc_last(parts, dpt, b):
    D = parts.shape[2]

    def body(p_ref, d_ref, b_ref, o_ref):
        norm = _norm_from(d_ref[...])
        p = p_ref[...]
        x = (p[0] + p[1]) * norm + b_ref[...]
        o_ref[...] = jnp.tanh(x)

    return pl.pallas_call(
        body,
        grid=(N_NODES // _BLK,),
        in_specs=[
            pl.BlockSpec((NC, _BLK, D), lambda i: (0, i, 0)),
            pl.BlockSpec((_BLK, 2), lambda i: (i, 0)),
            pl.BlockSpec((1, D), lambda i: (0, 0)),
        ],
        out_specs=pl.BlockSpec((_BLK, D), lambda i: (i, 0)),
        out_shape=jax.ShapeDtypeStruct((N_NODES, D), jnp.float32),
    )(parts, dpt, b)


def kernel(features, edge_index, W1, b1, W2, b2, W3, b3):
    src = edge_index[0].astype(jnp.int32)
    dst = edge_index[1].astype(jnp.int32)
    pad = E_PAD - N_EDGES
    ar = jnp.arange(pad, dtype=jnp.int32)
    src_p = jnp.concatenate([src, ar % N_NODES]).reshape(NW, NCH, B)
    trash = N_NODES + (ar % (N_ACC - N_NODES))
    dst_p = jnp.concatenate([dst, trash]).reshape(NW, NCH, B)

    deg_parts = _make_deg()(dst_p)
    dpt = deg_parts[:, :N_NODES, 0].T

    agg128 = _make_agg(128)
    agg48 = _make_agg(48)

    hw1 = _tc_first(features, W1, dpt)
    p1 = agg128(src_p, dst_p, hw1)
    hw2 = _tc_mid(p1, dpt, b1.reshape(1, 128), W2)
    p2 = agg128(src_p, dst_p, hw2)
    W3p = jnp.pad(W3, ((0, 0), (0, 8)))
    b3p = jnp.pad(b3, (0, 8)).reshape(1, 48)
    hw3 = _tc_mid(p2, dpt, b2.reshape(1, 128), W3p)
    p3 = agg48(src_p, dst_p, hw3)
    out48 = _tc_last(p3, dpt, b3p)
    return out48[:, :40]


# speedup vs baseline: 3.1592x; 1.0154x over previous
import functools

import jax
import jax.numpy as jnp
from jax import lax
from jax.experimental import pallas as pl
from jax.experimental.pallas import tpu as pltpu
from jax.experimental.pallas import tpu_sc as plsc

N_NODES = 10000
N_EDGES = 320000
NC = 2
NS = 16
NW = NC * NS
B = 112
NCH = 90
SEG = 6
SEGC = NCH // SEG
E_PAD = NW * NCH * B
ROWS_PER_TILE = 640
N_ACC = NS * ROWS_PER_TILE
ZROWS = 64


def _make_agg(D):
    mesh = plsc.VectorSubcoreMesh(core_axis_name="c", subcore_axis_name="s")

    @functools.partial(
        pl.kernel,
        mesh=mesh,
        out_type=jax.ShapeDtypeStruct((NC, N_ACC, D), jnp.float32),
        compiler_params=pltpu.CompilerParams(use_tc_tiling_on_sc=False),
        scratch_types=[
            pltpu.VMEM((SEGC, B), jnp.int32),
            pltpu.VMEM((SEGC, B), jnp.int32),
            pltpu.VMEM((B, D), jnp.float32),
            pltpu.VMEM((B, D), jnp.float32),
            pltpu.VMEM((B, D), jnp.float32),
            pltpu.VMEM_SHARED((N_ACC, D), jnp.float32),
            pltpu.SemaphoreType.DMA,
            pltpu.SemaphoreType.DMA,
            pltpu.SemaphoreType.DMA,
        ],
    )
    def agg(src_hbm, dst_hbm, h_hbm, out_hbm, src_v, dst_v, r0, r1, r2, acc,
            s0, s1, s2):
        c = lax.axis_index("c")
        s = lax.axis_index("s")
        wid = c * NS + s

        def _zb(k, _):
            i = k // (D // 16)
            j = k % (D // 16)
            r0[i, pl.ds(j * 16, 16)] = jnp.zeros((16,), jnp.float32)
            return 0
        lax.fori_loop(0, ZROWS * (D // 16), _zb, 0)

        def _zc(i, _):
            pltpu.sync_copy(
                r0.at[pl.ds(0, ZROWS)],
                acc.at[pl.ds(s * ROWS_PER_TILE + i * ZROWS, ZROWS)])
            return 0
        lax.fori_loop(0, ROWS_PER_TILE // ZROWS, _zc, 0)
        plsc.subcore_barrier()

        def _seg(sg, _):
            pltpu.sync_copy(src_hbm.at[wid, pl.ds(sg * SEGC, SEGC)], src_v)
            pltpu.sync_copy(dst_hbm.at[wid, pl.ds(sg * SEGC, SEGC)], dst_v)
            pltpu.async_copy(h_hbm.at[src_v.at[0]], r0, s0)
            pltpu.async_copy(h_hbm.at[src_v.at[1]], r1, s1)

            def _triple(t, _):
                j0 = 3 * t
                pltpu.make_async_copy(h_hbm.at[src_v.at[j0]], r0, s0).wait()
                pltpu.async_copy(h_hbm.at[src_v.at[j0 + 2]], r2, s2)
                pltpu.sync_copy(r0, acc.at[dst_v.at[j0]], add=True)

                pltpu.make_async_copy(h_hbm.at[src_v.at[j0 + 1]], r1, s1).wait()

                @pl.when(j0 + 3 < SEGC)
                def _():
                    pltpu.async_copy(h_hbm.at[src_v.at[j0 + 3]], r0, s0)
                pltpu.sync_copy(r1, acc.at[dst_v.at[j0 + 1]], add=True)

                pltpu.make_async_copy(h_hbm.at[src_v.at[j0 + 2]], r2, s2).wait()

                @pl.when(j0 + 4 < SEGC)
                def _():
                    pltpu.async_copy(h_hbm.at[src_v.at[j0 + 4]], r1, s1)
                pltpu.sync_copy(r2, acc.at[dst_v.at[j0 + 2]], add=True)
                return 0
            lax.fori_loop(0, SEGC // 3, _triple, 0)
            return 0
        lax.fori_loop(0, SEG, _seg, 0)
        plsc.subcore_barrier()

        pltpu.sync_copy(
            acc.at[pl.ds(s * ROWS_PER_TILE, ROWS_PER_TILE)],
            out_hbm.at[c, pl.ds(s * ROWS_PER_TILE, ROWS_PER_TILE)],
        )

    return agg


def _make_deg():
    D = 16
    mesh = plsc.VectorSubcoreMesh(core_axis_name="c", subcore_axis_name="s")

    @functools.partial(
        pl.kernel,
        mesh=mesh,
        out_type=jax.ShapeDtypeStruct((NC, N_ACC, D), jnp.float32),
        compiler_params=pltpu.CompilerParams(use_tc_tiling_on_sc=False),
        scratch_types=[
            pltpu.VMEM((NCH, B), jnp.int32),
            pltpu.VMEM((B, D), jnp.float32),
            pltpu.VMEM((ZROWS, D), jnp.float32),
            pltpu.VMEM_SHARED((N_ACC, D), jnp.float32),
        ],
    )
    def deg(dst_hbm, out_hbm, dst_v, ones_v, zbuf, acc):
        c = lax.axis_index("c")
        s = lax.axis_index("s")
        wid = c * NS + s

        def _init(i, _):
            ones_v[i] = jnp.ones((D,), jnp.float32)
            return 0
        lax.fori_loop(0, B, _init, 0)

        def _zb(i, _):
            zbuf[i] = jnp.zeros((D,), jnp.float32)
            return 0
        lax.fori_loop(0, ZROWS, _zb, 0)

        def _zc(i, _):
            pltpu.sync_copy(zbuf, acc.at[pl.ds(s * ROWS_PER_TILE + i * ZROWS, ZROWS)])
            return 0
        lax.fori_loop(0, ROWS_PER_TILE // ZROWS, _zc, 0)

        pltpu.sync_copy(dst_hbm.at[wid], dst_v)
        plsc.subcore_barrier()

        def _body(j, _):
            pltpu.sync_copy(ones_v, acc.at[dst_v.at[j]], add=True)
            return 0
        lax.fori_loop(0, NCH, _body, 0)
        plsc.subcore_barrier()

        pltpu.sync_copy(
            acc.at[pl.ds(s * ROWS_PER_TILE, ROWS_PER_TILE)],
            out_hbm.at[c, pl.ds(s * ROWS_PER_TILE, ROWS_PER_TILE)],
        )

    return deg


_BLK = 1000


def _norm_from(d_blk):
    deg = d_blk[:, 0:1] + d_blk[:, 1:2]
    return jnp.where(deg > 0, lax.rsqrt(deg), 0.0)


def _tc_first(feats, W, dpt):
    Dn = W.shape[1]

    def body(f_ref, w_ref, d_ref, o_ref):
        norm = _norm_from(d_ref[...])
        x = f_ref[...] * norm
        o_ref[...] = jnp.dot(x, w_ref[...], preferred_element_type=jnp.float32)

    return pl.pallas_call(
        body,
        grid=(N_NODES // _BLK,),
        in_specs=[
            pl.BlockSpec((_BLK, feats.shape[1]), lambda i: (i, 0)),
            pl.BlockSpec((feats.shape[1], Dn), lambda i: (0, 0)),
            pl.BlockSpec((_BLK, 2), lambda i: (i, 0)),
        ],
        out_specs=pl.BlockSpec((_BLK, Dn), lambda i: (i, 0)),
        out_shape=jax.ShapeDtypeStruct((N_NODES, Dn), jnp.float32),
    )(feats, W, dpt)


def _tc_mid(parts, dpt, b, W):
    D = parts.shape[2]
    Dn = W.shape[1]

    def body(p_ref, d_ref, b_ref, w_ref, o_ref):
        norm = _norm_from(d_ref[...])
        p = p_ref[...]
        x = (p[0] + p[1]) * norm + b_ref[...]
        x = jnp.maximum(x, 0.0) * norm
        o_ref[...] = jnp.dot(x, w_ref[...], preferred_element_type=jnp.float32)

    return pl.pallas_call(
        body,
        grid=(N_NODES // _BLK,),
        in_specs=[
            pl.BlockSpec((NC, _BLK, D), lambda i: (0, i, 0)),
            pl.BlockSpec((_BLK, 2), lambda i: (i, 0)),
            pl.BlockSpec((1, D), lambda i: (0, 0)),
            pl.BlockSpec((D, Dn), lambda i: (0, 0)),
        ],
        out_specs=pl.BlockSpec((_BLK, Dn), lambda i: (i, 0)),
        out_shape=jax.ShapeDtypeStruct((N_NODES, Dn), jnp.float32),
    )(parts, dpt, b, W)


def _tc_last(parts, dpt, b):
    D = parts.shape[2]

    def body(p_ref, d_ref, b_ref, o_ref):
        norm = _norm_from(d_ref[...])
        p = p_ref[...]
        x = (p[0] + p[1]) * norm + b_ref[...]
        o_ref[...] = jnp.tanh(x)

    return pl.pallas_call(
        body,
        grid=(N_NODES // _BLK,),
        in_specs=[
            pl.BlockSpec((NC, _BLK, D), lambda i: (0, i, 0)),
            pl.BlockSpec((_BLK, 2), lambda i: (i, 0)),
            pl.BlockSpec((1, D), lambda i: (0, 0)),
        ],
        out_specs=pl.BlockSpec((_BLK, D), lambda i: (i, 0)),
        out_shape=jax.ShapeDtypeStruct((N_NODES, D), jnp.float32),
    )(parts, dpt, b)


def kernel(features, edge_index, W1, b1, W2, b2, W3, b3):
    src = edge_index[0].astype(jnp.int32)
    dst = edge_index[1].astype(jnp.int32)
    pad = E_PAD - N_EDGES
    ar = jnp.arange(pad, dtype=jnp.int32)
    src_p = jnp.concatenate([src, ar % N_NODES]).reshape(NW, NCH, B)
    trash = N_NODES + (ar % (N_ACC - N_NODES))
    dst_p = jnp.concatenate([dst, trash]).reshape(NW, NCH, B)

    deg_parts = _make_deg()(dst_p)
    dpt = deg_parts[:, :N_NODES, 0].T

    agg128 = _make_agg(128)
    agg48 = _make_agg(48)

    hw1 = _tc_first(features, W1, dpt)
    p1 = agg128(src_p, dst_p, hw1)
    hw2 = _tc_mid(p1, dpt, b1.reshape(1, 128), W2)
    p2 = agg128(src_p, dst_p, hw2)
    W3p = jnp.pad(W3, ((0, 0), (0, 8)))
    b3p = jnp.pad(b3, (0, 8)).reshape(1, 48)
    hw3 = _tc_mid(p2, dpt, b2.reshape(1, 128), W3p)
    p3 = agg48(src_p, dst_p, hw3)
    out48 = _tc_last(p3, dpt, b3p)
    return out48[:, :40]
